# Initial kernel scaffold; baseline (speedup 1.0000x reference)
import jax
import jax.numpy as jnp
from jax.experimental import pallas as pl


def kernel(x, edge_index, edge_weight, W1, b1, W2, b2, W3, b3):
    raise NotImplementedError("write your pallas kernel here")


# speedup vs baseline: 4.8128x; 4.8128x over previous
import jax
import jax.numpy as jnp
from jax import lax
from jax.experimental import pallas as pl
from jax.experimental.pallas import tpu as pltpu
from jax.experimental.pallas import tpu_sc as plsc

N = 10000
E = 320000
D = 128
NTILES = 32
F = D // NTILES
L = 16
EC_DEG = E // NTILES
C_DEG = 2000
C_AGG = 4000


def _deg_body(dst_hbm, ew_hbm, out_hbm, dst_v, ew_v, acc):
    wid = lax.axis_index("s") * 2 + lax.axis_index("c")
    zero = jnp.zeros((L,), jnp.float32)

    def zb(i, _):
        acc[0, pl.ds(i * L, L)] = zero
        return 0

    lax.fori_loop(0, N // L, zb, 0)

    base = wid * EC_DEG
    zrow = jnp.zeros((L,), jnp.int32)

    def chunk(c, _):
        off = base + c * C_DEG
        pltpu.sync_copy(dst_hbm.at[pl.ds(off, C_DEG)], dst_v)
        pltpu.sync_copy(ew_hbm.at[pl.ds(off, C_DEG)], ew_v)

        def ib(i, _):
            idx = dst_v[pl.ds(i * L, L)]
            w = ew_v[pl.ds(i * L, L)]
            plsc.addupdate_scatter(acc, [zrow, idx], w)
            return 0

        lax.fori_loop(0, C_DEG // L, ib, 0)
        return 0

    lax.fori_loop(0, EC_DEG // C_DEG, chunk, 0)
    pltpu.sync_copy(acc, out_hbm.at[pl.ds(wid, 1)])


def _sc_deg(dst, ew):
    mesh = plsc.VectorSubcoreMesh(core_axis_name="c", subcore_axis_name="s")
    f = pl.kernel(
        _deg_body,
        out_type=jax.ShapeDtypeStruct((NTILES, N), jnp.float32),
        mesh=mesh,
        compiler_params=pltpu.CompilerParams(needs_layout_passes=False),
        scratch_types=[
            pltpu.VMEM((C_DEG,), jnp.int32),
            pltpu.VMEM((C_DEG,), jnp.float32),
            pltpu.VMEM((1, N), jnp.float32),
        ],
    )
    return f(dst, ew)


def _agg_body(hp_hbm, src_hbm, dst_hbm, ew_hbm, out_hbm, h_v, acc, src_v,
              dst_v, ew_v):
    wid = lax.axis_index("s") * 2 + lax.axis_index("c")
    r0 = wid * F
    pltpu.sync_copy(hp_hbm.at[pl.ds(r0, F)], h_v)

    zero = jnp.zeros((L,), jnp.float32)

    def zb(i, _):
        for f in range(F):
            acc[f, pl.ds(i * L, L)] = zero
        return 0

    lax.fori_loop(0, N // L, zb, 0)

    def chunk(c, _):
        off = c * C_AGG
        pltpu.sync_copy(src_hbm.at[pl.ds(off, C_AGG)], src_v)
        pltpu.sync_copy(dst_hbm.at[pl.ds(off, C_AGG)], dst_v)
        pltpu.sync_copy(ew_hbm.at[pl.ds(off, C_AGG)], ew_v)

        def ib(i, _):
            s = src_v[pl.ds(i * L, L)]
            d = dst_v[pl.ds(i * L, L)]
            w = ew_v[pl.ds(i * L, L)]
            for f in range(F):
                fv = jnp.full((L,), f, jnp.int32)
                v = plsc.load_gather(h_v, [fv, s])
                plsc.addupdate_scatter(acc, [fv, d], v * w)
            return 0

        lax.fori_loop(0, C_AGG // L, ib, 0)
        return 0

    lax.fori_loop(0, E // C_AGG, chunk, 0)
    pltpu.sync_copy(acc, out_hbm.at[pl.ds(r0, F)])


def _sc_agg(hp, src, dst, ew):
    mesh = plsc.VectorSubcoreMesh(core_axis_name="c", subcore_axis_name="s")
    f = pl.kernel(
        _agg_body,
        out_type=jax.ShapeDtypeStruct((D, N), jnp.float32),
        mesh=mesh,
        compiler_params=pltpu.CompilerParams(needs_layout_passes=False),
        scratch_types=[
            pltpu.VMEM((F, N), jnp.float32),
            pltpu.VMEM((F, N), jnp.float32),
            pltpu.VMEM((C_AGG,), jnp.int32),
            pltpu.VMEM((C_AGG,), jnp.int32),
            pltpu.VMEM((C_AGG,), jnp.float32),
        ],
    )
    return f(hp, src, dst, ew)


def _tc_first_body(dp_ref, xT_ref, W_ref, b_ref, hp_ref, dis_ref):
    deg = jnp.sum(dp_ref[...], axis=0, keepdims=True) + 1.0
    dis = lax.rsqrt(deg)
    h = lax.dot_general(W_ref[...], xT_ref[...], (((0,), (0,)), ((), ())),
                        preferred_element_type=jnp.float32)
    hp_ref[...] = (h + b_ref[...]) * dis
    dis_ref[...] = dis


def _tc_first(dp, xT, W, bcol):
    return pl.pallas_call(
        _tc_first_body,
        out_shape=(jax.ShapeDtypeStruct((D, N), jnp.float32),
                   jax.ShapeDtypeStruct((1, N), jnp.float32)),
    )(dp, xT, W, bcol)


def _tc_mid_body(acc_ref, hp_ref, dis_ref, W_ref, b_ref, out_ref):
    dis = dis_ref[...]
    act = jnp.maximum(dis * (acc_ref[...] + hp_ref[...]), 0.0)
    h = lax.dot_general(W_ref[...], act, (((0,), (0,)), ((), ())),
                        preferred_element_type=jnp.float32)
    out_ref[...] = (h + b_ref[...]) * dis


def _tc_mid(acc, hp, dis, W, bcol):
    return pl.pallas_call(
        _tc_mid_body,
        out_shape=jax.ShapeDtypeStruct((D, N), jnp.float32),
    )(acc, hp, dis, W, bcol)


def _tc_last_body(acc_ref, hp_ref, dis_ref, out_ref):
    out_ref[...] = jnp.maximum(dis_ref[...] * (acc_ref[...] + hp_ref[...]),
                               0.0)


def _tc_last(acc, hp, dis):
    return pl.pallas_call(
        _tc_last_body,
        out_shape=jax.ShapeDtypeStruct((D, N), jnp.float32),
    )(acc, hp, dis)


def kernel(x, edge_index, edge_weight, W1, b1, W2, b2, W3, b3):
    src = edge_index[0]
    dst = edge_index[1]
    xT = x.T
    dp = _sc_deg(dst, edge_weight)
    h1p, dis = _tc_first(dp, xT, W1, b1.reshape(D, 1))
    acc1 = _sc_agg(h1p, src, dst, edge_weight)
    h2p = _tc_mid(acc1, h1p, dis, W2, b2.reshape(D, 1))
    acc2 = _sc_agg(h2p, src, dst, edge_weight)
    h3p = _tc_mid(acc2, h2p, dis, W3, b3.reshape(D, 1))
    acc3 = _sc_agg(h3p, src, dst, edge_weight)
    outT = _tc_last(acc3, h3p, dis)
    return outT.T


# speedup vs baseline: 6.0029x; 1.2473x over previous
import jax
import jax.numpy as jnp
from jax import lax
from jax.experimental import pallas as pl
from jax.experimental.pallas import tpu as pltpu
from jax.experimental.pallas import tpu_sc as plsc

N = 10000
E = 320000
D = 128
NTILES = 32
F = D // NTILES
L = 16
EC_DEG = E // NTILES
C_DEG = 2000
C_AGG = 6400
UNROLL = 4


def _deg_body(dst_hbm, ew_hbm, out_hbm, dst_v, ew_v, acc):
    wid = lax.axis_index("s") * 2 + lax.axis_index("c")
    zero = jnp.zeros((L,), jnp.float32)

    def zb(i, _):
        acc[0, pl.ds(i * L, L)] = zero
        return 0

    lax.fori_loop(0, N // L, zb, 0)

    base = wid * EC_DEG
    zrow = jnp.zeros((L,), jnp.int32)

    def chunk(c, _):
        off = base + c * C_DEG
        pltpu.sync_copy(dst_hbm.at[pl.ds(off, C_DEG)], dst_v)
        pltpu.sync_copy(ew_hbm.at[pl.ds(off, C_DEG)], ew_v)

        def ib(i, _):
            idx = dst_v[pl.ds(i * L, L)]
            w = ew_v[pl.ds(i * L, L)]
            plsc.addupdate_scatter(acc, [zrow, idx], w)
            return 0

        lax.fori_loop(0, C_DEG // L, ib, 0)
        return 0

    lax.fori_loop(0, EC_DEG // C_DEG, chunk, 0)
    pltpu.sync_copy(acc, out_hbm.at[pl.ds(wid, 1)])


def _sc_deg(dst, ew):
    mesh = plsc.VectorSubcoreMesh(core_axis_name="c", subcore_axis_name="s")
    f = pl.kernel(
        _deg_body,
        out_type=jax.ShapeDtypeStruct((NTILES, N), jnp.float32),
        mesh=mesh,
        compiler_params=pltpu.CompilerParams(needs_layout_passes=False),
        scratch_types=[
            pltpu.VMEM((C_DEG,), jnp.int32),
            pltpu.VMEM((C_DEG,), jnp.float32),
            pltpu.VMEM((1, N), jnp.float32),
        ],
    )
    return f(dst, ew)


def _agg_body(hp_hbm, ei_hbm, ew_hbm, out_hbm, h_v, acc, sd0, sd1, ew0, ew1,
              sem0, sem1):
    wid = lax.axis_index("s") * 2 + lax.axis_index("c")
    r0 = wid * F
    pltpu.sync_copy(hp_hbm.at[pl.ds(r0, F)], h_v)

    zero = jnp.zeros((L,), jnp.float32)

    def zb(i, _):
        for f in range(F):
            acc[f, pl.ds(i * L, L)] = zero
        return 0

    lax.fori_loop(0, N // L, zb, 0)

    sd = (sd0, sd1)
    ewb = (ew0, ew1)
    sems = (sem0, sem1)
    nchunks = E // C_AGG
    last_off = (nchunks - 1) * C_AGG

    def issue(c, b):
        off = jnp.minimum(c * C_AGG, last_off)
        ha = pltpu.async_copy(ei_hbm.at[:, pl.ds(off, C_AGG)], sd[b], sems[b])
        hb = pltpu.async_copy(ew_hbm.at[pl.ds(off, C_AGG)], ewb[b], sems[b])
        return ha, hb

    def wait(b):
        pltpu.make_async_copy(ei_hbm.at[:, pl.ds(0, C_AGG)], sd[b],
                              sems[b]).wait()
        pltpu.make_async_copy(ew_hbm.at[pl.ds(0, C_AGG)], ewb[b],
                              sems[b]).wait()

    issue(0, 0)
    issue(1, 1)

    def compute(b):
        wait(b)

        def ib(i, _):
            for u in range(UNROLL):
                base = (i * UNROLL + u) * L
                s = sd[b][0, pl.ds(base, L)]
                d = sd[b][1, pl.ds(base, L)]
                w = ewb[b][pl.ds(base, L)]
                for f in range(F):
                    fv = jnp.full((L,), f, jnp.int32)
                    v = plsc.load_gather(h_v, [fv, s])
                    plsc.addupdate_scatter(acc, [fv, d], v * w)
            return 0

        lax.fori_loop(0, C_AGG // (L * UNROLL), ib, 0)

    def pair(p, _):
        c = p * 2
        compute(0)
        issue(c + 2, 0)
        compute(1)
        issue(c + 3, 1)
        return 0

    lax.fori_loop(0, nchunks // 2, pair, 0)
    wait(0)
    wait(1)
    pltpu.sync_copy(acc, out_hbm.at[pl.ds(r0, F)])


def _sc_agg(hp, ei, ew):
    mesh = plsc.VectorSubcoreMesh(core_axis_name="c", subcore_axis_name="s")
    f = pl.kernel(
        _agg_body,
        out_type=jax.ShapeDtypeStruct((D, N), jnp.float32),
        mesh=mesh,
        compiler_params=pltpu.CompilerParams(needs_layout_passes=False),
        scratch_types=[
            pltpu.VMEM((F, N), jnp.float32),
            pltpu.VMEM((F, N), jnp.float32),
            pltpu.VMEM((2, C_AGG), jnp.int32),
            pltpu.VMEM((2, C_AGG), jnp.int32),
            pltpu.VMEM((C_AGG,), jnp.float32),
            pltpu.VMEM((C_AGG,), jnp.float32),
            pltpu.SemaphoreType.DMA,
            pltpu.SemaphoreType.DMA,
        ],
    )
    return f(hp, ei, ew)


def _tc_first_body(dp_ref, xT_ref, W_ref, b_ref, hp_ref, dis_ref):
    deg = jnp.sum(dp_ref[...], axis=0, keepdims=True) + 1.0
    dis = lax.rsqrt(deg)
    h = lax.dot_general(W_ref[...], xT_ref[...], (((0,), (0,)), ((), ())),
                        preferred_element_type=jnp.float32)
    hp_ref[...] = (h + b_ref[...]) * dis
    dis_ref[...] = dis


def _tc_first(dp, xT, W, bcol):
    return pl.pallas_call(
        _tc_first_body,
        out_shape=(jax.ShapeDtypeStruct((D, N), jnp.float32),
                   jax.ShapeDtypeStruct((1, N), jnp.float32)),
    )(dp, xT, W, bcol)


def _tc_mid_body(acc_ref, hp_ref, dis_ref, W_ref, b_ref, out_ref):
    dis = dis_ref[...]
    act = jnp.maximum(dis * (acc_ref[...] + hp_ref[...]), 0.0)
    h = lax.dot_general(W_ref[...], act, (((0,), (0,)), ((), ())),
                        preferred_element_type=jnp.float32)
    out_ref[...] = (h + b_ref[...]) * dis


def _tc_mid(acc, hp, dis, W, bcol):
    return pl.pallas_call(
        _tc_mid_body,
        out_shape=jax.ShapeDtypeStruct((D, N), jnp.float32),
    )(acc, hp, dis, W, bcol)


def _tc_last_body(acc_ref, hp_ref, dis_ref, out_ref):
    out_ref[...] = jnp.maximum(dis_ref[...] * (acc_ref[...] + hp_ref[...]),
                               0.0)


def _tc_last(acc, hp, dis):
    return pl.pallas_call(
        _tc_last_body,
        out_shape=jax.ShapeDtypeStruct((D, N), jnp.float32),
    )(acc, hp, dis)


def kernel(x, edge_index, edge_weight, W1, b1, W2, b2, W3, b3):
    dst = edge_index[1]
    xT = x.T
    dp = _sc_deg(dst, edge_weight)
    h1p, dis = _tc_first(dp, xT, W1, b1.reshape(D, 1))
    acc1 = _sc_agg(h1p, edge_index, edge_weight)
    h2p = _tc_mid(acc1, h1p, dis, W2, b2.reshape(D, 1))
    acc2 = _sc_agg(h2p, edge_index, edge_weight)
    h3p = _tc_mid(acc2, h2p, dis, W3, b3.reshape(D, 1))
    acc3 = _sc_agg(h3p, edge_index, edge_weight)
    outT = _tc_last(acc3, h3p, dis)
    return outT.T


# speedup vs baseline: 9.4747x; 1.5784x over previous
import jax
import jax.numpy as jnp
from jax import lax
from jax.experimental import pallas as pl
from jax.experimental.pallas import tpu as pltpu
from jax.experimental.pallas import tpu_sc as plsc

N = 10000
E = 320000
D = 128
NTILES = 32
L = 16
EC_DEG = E // NTILES
C_DEG = 2000
CE = 40
TE = E // NTILES
NCH = TE // CE
RPT = 624
TAIL = N - 15 * RPT


def _deg_body(dst_hbm, ew_hbm, out_hbm, dst_v, ew_v, acc):
    wid = lax.axis_index("s") * 2 + lax.axis_index("c")
    zero = jnp.zeros((L,), jnp.float32)

    def zb(i, _):
        acc[0, pl.ds(i * L, L)] = zero
        return 0

    lax.fori_loop(0, N // L, zb, 0)

    base = wid * EC_DEG
    zrow = jnp.zeros((L,), jnp.int32)

    def chunk(c, _):
        off = base + c * C_DEG
        pltpu.sync_copy(dst_hbm.at[pl.ds(off, C_DEG)], dst_v)
        pltpu.sync_copy(ew_hbm.at[pl.ds(off, C_DEG)], ew_v)

        def ib(i, _):
            idx = dst_v[pl.ds(i * L, L)]
            w = ew_v[pl.ds(i * L, L)]
            plsc.addupdate_scatter(acc, [zrow, idx], w)
            return 0

        lax.fori_loop(0, C_DEG // L, ib, 0)
        return 0

    lax.fori_loop(0, EC_DEG // C_DEG, chunk, 0)
    pltpu.sync_copy(acc, out_hbm.at[pl.ds(wid, 1)])


def _sc_deg(dst, ew):
    mesh = plsc.VectorSubcoreMesh(core_axis_name="c", subcore_axis_name="s")
    f = pl.kernel(
        _deg_body,
        out_type=jax.ShapeDtypeStruct((NTILES, N), jnp.float32),
        mesh=mesh,
        compiler_params=pltpu.CompilerParams(needs_layout_passes=False),
        scratch_types=[
            pltpu.VMEM((C_DEG,), jnp.int32),
            pltpu.VMEM((C_DEG,), jnp.float32),
            pltpu.VMEM((1, N), jnp.float32),
        ],
    )
    return f(dst, ew)


def _agg_body(hp_hbm, src_hbm, dst_hbm, ew_hbm, out_hbm,
              rows0, rows1, didx0, didx1, sidx_big, ew_big,
              shared, semi0, semi1, semg0, semg1):
    cid = lax.axis_index("c")
    sid = lax.axis_index("s")
    wid = sid * 2 + cid
    base = wid * TE

    rows = (rows0, rows1)
    didx = (didx0, didx1)
    semi = (semi0, semi1)
    semg = (semg0, semg1)

    zero = jnp.zeros((L,), jnp.float32)

    def zb(j, _):
        for k in range(D // L):
            rows0[j, pl.ds(k * L, L)] = zero
        return 0

    lax.fori_loop(0, CE, zb, 0)
    r0 = sid * RPT
    for k in range(RPT // CE):
        pltpu.sync_copy(rows0, shared.at[pl.ds(r0 + k * CE, CE)])
    rem = RPT - (RPT // CE) * CE
    pltpu.sync_copy(rows0.at[pl.ds(0, rem)],
                    shared.at[pl.ds(r0 + (RPT // CE) * CE, rem)])

    @pl.when(sid == 15)
    def _():
        pltpu.sync_copy(rows0.at[pl.ds(0, TAIL - RPT)],
                        shared.at[pl.ds(16 * RPT, TAIL - RPT)])

    plsc.subcore_barrier()

    pltpu.sync_copy(src_hbm.at[pl.ds(base, TE)], sidx_big)
    pltpu.sync_copy(ew_hbm.at[pl.ds(base, TE)], ew_big.at[pl.ds(0, TE)])

    last = (NCH - 1) * CE

    def issue_didx(b, c):
        off = base + jnp.minimum(c * CE, last)
        return pltpu.async_copy(dst_hbm.at[pl.ds(off, CE)], didx[b].at[0],
                                semi[b])

    def wait_didx(b):
        pltpu.make_async_copy(dst_hbm.at[pl.ds(0, CE)], didx[b].at[0],
                              semi[b]).wait()

    def issue_gather(b, c):
        off = jnp.minimum(c * CE, last)
        return pltpu.async_copy(hp_hbm.at[sidx_big.at[pl.ds(off, CE)]],
                                rows[b], semg[b])

    def wait_gather(b):
        pltpu.make_async_copy(hp_hbm.at[sidx_big.at[pl.ds(0, CE)]], rows[b],
                              semg[b]).wait()

    def scale(b, c):
        rb = rows[b]

        def sg(g, _):
            wv = ew_big[pl.ds(c * CE + g * 8, L)]
            for u in range(8):
                wvec = jnp.full((L,), wv[u], jnp.float32)
                j = g * 8 + u
                for k in range(D // L):
                    rb[j, pl.ds(k * L, L)] = rb[j, pl.ds(k * L, L)] * wvec
            return 0

        lax.fori_loop(0, CE // 8, sg, 0)

    def scatter(b):
        pltpu.sync_copy(rows[b], shared.at[didx[b].at[0]], add=True)

    issue_didx(0, 0)
    issue_didx(1, 1)
    issue_gather(0, 0)
    issue_gather(1, 1)

    def pair(p, _):
        c = p * 2
        for b in range(2):
            wait_gather(b)
            scale(b, c + b)
            wait_didx(b)
            scatter(b)
            issue_didx(b, c + 2 + b)
            issue_gather(b, c + 2 + b)
        return 0

    lax.fori_loop(0, NCH // 2, pair, 0)
    wait_gather(0)
    wait_gather(1)
    wait_didx(0)
    wait_didx(1)

    plsc.subcore_barrier()

    pltpu.sync_copy(shared.at[pl.ds(r0, RPT)],
                    out_hbm.at[cid, pl.ds(r0, RPT)])

    @pl.when(sid == 15)
    def _():
        pltpu.sync_copy(shared.at[pl.ds(16 * RPT, TAIL - RPT)],
                        out_hbm.at[cid, pl.ds(16 * RPT, TAIL - RPT)])


def _sc_agg(hp, src, dst, ew):
    mesh = plsc.VectorSubcoreMesh(core_axis_name="c", subcore_axis_name="s")
    f = pl.kernel(
        _agg_body,
        out_type=jax.ShapeDtypeStruct((2, N, D), jnp.float32),
        mesh=mesh,
        compiler_params=pltpu.CompilerParams(needs_layout_passes=False),
        scratch_types=[
            pltpu.VMEM((CE, D), jnp.float32),
            pltpu.VMEM((CE, D), jnp.float32),
            pltpu.VMEM((1, CE), jnp.int32),
            pltpu.VMEM((1, CE), jnp.int32),
            pltpu.VMEM((TE,), jnp.int32),
            pltpu.VMEM((TE + L,), jnp.float32),
            pltpu.VMEM_SHARED((N, D), jnp.float32),
            pltpu.SemaphoreType.DMA,
            pltpu.SemaphoreType.DMA,
            pltpu.SemaphoreType.DMA,
            pltpu.SemaphoreType.DMA,
        ],
    )
    return f(hp, src, dst, ew)


def _tc_first_body(dp_ref, x_ref, W_ref, b_ref, hp_ref, dis_ref):
    ones = jnp.ones((NTILES, 1), jnp.float32)
    deg = lax.dot_general(dp_ref[...], ones, (((0,), (0,)), ((), ())),
                          preferred_element_type=jnp.float32) + 1.0
    dis = lax.rsqrt(deg)
    h = lax.dot_general(x_ref[...], W_ref[...], (((1,), (0,)), ((), ())),
                        preferred_element_type=jnp.float32)
    hp_ref[...] = (h + b_ref[...]) * dis
    dis_ref[...] = dis


def _tc_first(dp, x, W, brow):
    return pl.pallas_call(
        _tc_first_body,
        out_shape=(jax.ShapeDtypeStruct((N, D), jnp.float32),
                   jax.ShapeDtypeStruct((N, 1), jnp.float32)),
    )(dp, x, W, brow)


def _tc_mid_body(p2_ref, hp_ref, dis_ref, W_ref, b_ref, out_ref):
    dis = dis_ref[...]
    acc = p2_ref[0] + p2_ref[1]
    act = jnp.maximum(dis * (acc + hp_ref[...]), 0.0)
    h = lax.dot_general(act, W_ref[...], (((1,), (0,)), ((), ())),
                        preferred_element_type=jnp.float32)
    out_ref[...] = (h + b_ref[...]) * dis


def _tc_mid(p2, hp, dis, W, brow):
    return pl.pallas_call(
        _tc_mid_body,
        out_shape=jax.ShapeDtypeStruct((N, D), jnp.float32),
    )(p2, hp, dis, W, brow)


def _tc_last_body(p2_ref, hp_ref, dis_ref, out_ref):
    acc = p2_ref[0] + p2_ref[1]
    out_ref[...] = jnp.maximum(dis_ref[...] * (acc + hp_ref[...]), 0.0)


def _tc_last(p2, hp, dis):
    return pl.pallas_call(
        _tc_last_body,
        out_shape=jax.ShapeDtypeStruct((N, D), jnp.float32),
    )(p2, hp, dis)


def kernel(x, edge_index, edge_weight, W1, b1, W2, b2, W3, b3):
    src = edge_index[0]
    dst = edge_index[1]
    dp = _sc_deg(dst, edge_weight)
    h1p, dis = _tc_first(dp, x, W1, b1.reshape(1, D))
    p1 = _sc_agg(h1p, src, dst, edge_weight)
    h2p = _tc_mid(p1, h1p, dis, W2, b2.reshape(1, D))
    p2 = _sc_agg(h2p, src, dst, edge_weight)
    h3p = _tc_mid(p2, h2p, dis, W3, b3.reshape(1, D))
    p3 = _sc_agg(h3p, src, dst, edge_weight)
    return _tc_last(p3, h3p, dis)


# speedup vs baseline: 11.0175x; 1.1628x over previous
import jax
import jax.numpy as jnp
from jax import lax
from jax.experimental import pallas as pl
from jax.experimental.pallas import tpu as pltpu
from jax.experimental.pallas import tpu_sc as plsc

N = 10000
E = 320000
D = 128
NTILES = 32
L = 16
EC_DEG = E // NTILES
C_DEG = 2000
CE = 40
TE = E // NTILES
NCH = TE // CE
RPT = 624
TAIL = N - 15 * RPT


def _deg_body(dst_hbm, ew_hbm, out_hbm, dst_v, ew_v, acc):
    wid = lax.axis_index("s") * 2 + lax.axis_index("c")
    zero = jnp.zeros((L,), jnp.float32)

    def zb(i, _):
        acc[0, pl.ds(i * L, L)] = zero
        return 0

    lax.fori_loop(0, N // L, zb, 0)

    base = wid * EC_DEG
    zrow = jnp.zeros((L,), jnp.int32)

    def chunk(c, _):
        off = base + c * C_DEG
        pltpu.sync_copy(dst_hbm.at[pl.ds(off, C_DEG)], dst_v)
        pltpu.sync_copy(ew_hbm.at[pl.ds(off, C_DEG)], ew_v)

        def ib(i, _):
            idx = dst_v[pl.ds(i * L, L)]
            w = ew_v[pl.ds(i * L, L)]
            plsc.addupdate_scatter(acc, [zrow, idx], w)
            return 0

        lax.fori_loop(0, C_DEG // L, ib, 0)
        return 0

    lax.fori_loop(0, EC_DEG // C_DEG, chunk, 0)
    pltpu.sync_copy(acc, out_hbm.at[pl.ds(wid, 1)])


def _sc_deg(dst, ew):
    mesh = plsc.VectorSubcoreMesh(core_axis_name="c", subcore_axis_name="s")
    f = pl.kernel(
        _deg_body,
        out_type=jax.ShapeDtypeStruct((NTILES, N), jnp.float32),
        mesh=mesh,
        compiler_params=pltpu.CompilerParams(needs_layout_passes=False),
        scratch_types=[
            pltpu.VMEM((C_DEG,), jnp.int32),
            pltpu.VMEM((C_DEG,), jnp.float32),
            pltpu.VMEM((1, N), jnp.float32),
        ],
    )
    return f(dst, ew)


def _agg_body(hp_hbm, src_hbm, dst_hbm, ew_hbm, out_hbm,
              rows0, rows1, rows2, rows3, didx0, didx1, didx2, didx3,
              sidx_big, ew_big, shared,
              semi0, semi1, semi2, semi3,
              semg0, semg1, semg2, semg3,
              sems0, sems1, sems2, sems3):
    cid = lax.axis_index("c")
    sid = lax.axis_index("s")
    wid = sid * 2 + cid
    base = wid * TE

    rows = (rows0, rows1, rows2, rows3)
    didx = (didx0, didx1, didx2, didx3)
    semi = (semi0, semi1, semi2, semi3)
    semg = (semg0, semg1, semg2, semg3)
    sems = (sems0, sems1, sems2, sems3)

    zero = jnp.zeros((L,), jnp.float32)

    def zb(j, _):
        for k in range(D // L):
            rows0[j, pl.ds(k * L, L)] = zero
        return 0

    lax.fori_loop(0, CE, zb, 0)
    r0 = sid * RPT
    for k in range(RPT // CE):
        pltpu.sync_copy(rows0, shared.at[pl.ds(r0 + k * CE, CE)])
    rem = RPT - (RPT // CE) * CE
    pltpu.sync_copy(rows0.at[pl.ds(0, rem)],
                    shared.at[pl.ds(r0 + (RPT // CE) * CE, rem)])

    @pl.when(sid == 15)
    def _():
        pltpu.sync_copy(rows0.at[pl.ds(0, TAIL - RPT)],
                        shared.at[pl.ds(16 * RPT, TAIL - RPT)])

    plsc.subcore_barrier()

    pltpu.sync_copy(src_hbm.at[pl.ds(base, TE)], sidx_big)
    pltpu.sync_copy(ew_hbm.at[pl.ds(base, TE)], ew_big.at[pl.ds(0, TE)])

    last = (NCH - 1) * CE

    def issue_didx(b, c):
        off = base + jnp.minimum(c * CE, last)
        return pltpu.async_copy(dst_hbm.at[pl.ds(off, CE)], didx[b].at[0],
                                semi[b])

    def wait_didx(b):
        pltpu.make_async_copy(dst_hbm.at[pl.ds(0, CE)], didx[b].at[0],
                              semi[b]).wait()

    def issue_gather(b, c):
        off = jnp.minimum(c * CE, last)
        return pltpu.async_copy(hp_hbm.at[sidx_big.at[pl.ds(off, CE)]],
                                rows[b], semg[b])

    def wait_gather(b):
        pltpu.make_async_copy(hp_hbm.at[sidx_big.at[pl.ds(0, CE)]], rows[b],
                              semg[b]).wait()

    def scale(b, c):
        rb = rows[b]

        def sg(g, _):
            wv = ew_big[pl.ds(c * CE + g * 8, L)]
            for u in range(8):
                wvec = jnp.full((L,), wv[u], jnp.float32)
                j = g * 8 + u
                for k in range(D // L):
                    rb[j, pl.ds(k * L, L)] = rb[j, pl.ds(k * L, L)] * wvec
            return 0

        lax.fori_loop(0, CE // 8, sg, 0)

    def issue_scatter(b):
        return pltpu.async_copy(rows[b], shared.at[didx[b].at[0]], sems[b],
                                add=True)

    def wait_scatter(b):
        pltpu.make_async_copy(rows[b], shared.at[didx[b].at[0]],
                              sems[b]).wait()

    issue_didx(0, 0)
    issue_didx(1, 1)
    issue_gather(0, 0)
    issue_gather(1, 1)
    for b in range(2):
        wait_gather(b)
        scale(b, b)
        wait_didx(b)
        issue_scatter(b)
        issue_didx(b + 2, b + 2)
        issue_gather(b + 2, b + 2)

    def quad(p, _):
        c = p * 4 + 2
        for u in range(4):
            b = (2 + u) % 4
            bn = (b + 2) % 4
            wait_gather(b)
            scale(b, c + u)
            wait_didx(b)
            issue_scatter(b)
            wait_scatter(bn)
            issue_didx(bn, c + u + 2)
            issue_gather(bn, c + u + 2)
        return 0

    lax.fori_loop(0, (NCH - 2) // 4, quad, 0)
    wait_gather(NCH % 4)
    wait_gather((NCH + 1) % 4)
    wait_didx(NCH % 4)
    wait_didx((NCH + 1) % 4)
    wait_scatter((NCH - 2) % 4)
    wait_scatter((NCH - 1) % 4)

    plsc.subcore_barrier()

    pltpu.sync_copy(shared.at[pl.ds(r0, RPT)],
                    out_hbm.at[cid, pl.ds(r0, RPT)])

    @pl.when(sid == 15)
    def _():
        pltpu.sync_copy(shared.at[pl.ds(16 * RPT, TAIL - RPT)],
                        out_hbm.at[cid, pl.ds(16 * RPT, TAIL - RPT)])


def _sc_agg(hp, src, dst, ew):
    mesh = plsc.VectorSubcoreMesh(core_axis_name="c", subcore_axis_name="s")
    f = pl.kernel(
        _agg_body,
        out_type=jax.ShapeDtypeStruct((2, N, D), jnp.float32),
        mesh=mesh,
        compiler_params=pltpu.CompilerParams(needs_layout_passes=False),
        scratch_types=(
            [pltpu.VMEM((CE, D), jnp.float32)] * 4
            + [pltpu.VMEM((1, CE), jnp.int32)] * 4
            + [pltpu.VMEM((TE,), jnp.int32),
               pltpu.VMEM((TE + L,), jnp.float32),
               pltpu.VMEM_SHARED((N, D), jnp.float32)]
            + [pltpu.SemaphoreType.DMA] * 12
        ),
    )
    return f(hp, src, dst, ew)


def _tc_first_body(dp_ref, x_ref, W_ref, b_ref, hp_ref, dis_ref):
    ones = jnp.ones((NTILES, 1), jnp.float32)
    deg = lax.dot_general(dp_ref[...], ones, (((0,), (0,)), ((), ())),
                          preferred_element_type=jnp.float32) + 1.0
    dis = lax.rsqrt(deg)
    h = lax.dot_general(x_ref[...], W_ref[...], (((1,), (0,)), ((), ())),
                        preferred_element_type=jnp.float32)
    hp_ref[...] = (h + b_ref[...]) * dis
    dis_ref[...] = dis


def _tc_first(dp, x, W, brow):
    return pl.pallas_call(
        _tc_first_body,
        out_shape=(jax.ShapeDtypeStruct((N, D), jnp.float32),
                   jax.ShapeDtypeStruct((N, 1), jnp.float32)),
    )(dp, x, W, brow)


def _tc_mid_body(p2_ref, hp_ref, dis_ref, W_ref, b_ref, out_ref):
    dis = dis_ref[...]
    acc = p2_ref[0] + p2_ref[1]
    act = jnp.maximum(dis * (acc + hp_ref[...]), 0.0)
    h = lax.dot_general(act, W_ref[...], (((1,), (0,)), ((), ())),
                        preferred_element_type=jnp.float32)
    out_ref[...] = (h + b_ref[...]) * dis


def _tc_mid(p2, hp, dis, W, brow):
    return pl.pallas_call(
        _tc_mid_body,
        out_shape=jax.ShapeDtypeStruct((N, D), jnp.float32),
    )(p2, hp, dis, W, brow)


def _tc_last_body(p2_ref, hp_ref, dis_ref, out_ref):
    acc = p2_ref[0] + p2_ref[1]
    out_ref[...] = jnp.maximum(dis_ref[...] * (acc + hp_ref[...]), 0.0)


def _tc_last(p2, hp, dis):
    return pl.pallas_call(
        _tc_last_body,
        out_shape=jax.ShapeDtypeStruct((N, D), jnp.float32),
    )(p2, hp, dis)


def kernel(x, edge_index, edge_weight, W1, b1, W2, b2, W3, b3):
    src = edge_index[0]
    dst = edge_index[1]
    dp = _sc_deg(dst, edge_weight)
    h1p, dis = _tc_first(dp, x, W1, b1.reshape(1, D))
    p1 = _sc_agg(h1p, src, dst, edge_weight)
    h2p = _tc_mid(p1, h1p, dis, W2, b2.reshape(1, D))
    p2 = _sc_agg(h2p, src, dst, edge_weight)
    h3p = _tc_mid(p2, h2p, dis, W3, b3.reshape(1, D))
    p3 = _sc_agg(h3p, src, dst, edge_weight)
    return _tc_last(p3, h3p, dis)


# speedup vs baseline: 21.7474x; 1.9739x over previous
import jax
import jax.numpy as jnp
from jax import lax
from jax.experimental import pallas as pl
from jax.experimental.pallas import tpu as pltpu
from jax.experimental.pallas import tpu_sc as plsc

N = 10000
E = 320000
D = 128
NTILES = 32
L = 16
EC_DEG = E // NTILES
C_DEG = 2000
CE = 40
TE = E // NTILES
NCH = TE // CE
RPT = 624
TAIL = N - 15 * RPT


def _deg_body(dst_hbm, ew_hbm, out_hbm, dst_v, ew_v, acc):
    wid = lax.axis_index("s") * 2 + lax.axis_index("c")
    zero = jnp.zeros((L,), jnp.float32)

    def zb(i, _):
        acc[0, pl.ds(i * L, L)] = zero
        return 0

    lax.fori_loop(0, N // L, zb, 0)

    base = wid * EC_DEG
    zrow = jnp.zeros((L,), jnp.int32)

    def chunk(c, _):
        off = base + c * C_DEG
        pltpu.sync_copy(dst_hbm.at[pl.ds(off, C_DEG)], dst_v)
        pltpu.sync_copy(ew_hbm.at[pl.ds(off, C_DEG)], ew_v)

        def ib(i, _):
            idx = dst_v[pl.ds(i * L, L)]
            w = ew_v[pl.ds(i * L, L)]
            plsc.addupdate_scatter(acc, [zrow, idx], w)
            return 0

        lax.fori_loop(0, C_DEG // L, ib, 0)
        return 0

    lax.fori_loop(0, EC_DEG // C_DEG, chunk, 0)
    pltpu.sync_copy(acc, out_hbm.at[pl.ds(wid, 1)])


def _sc_deg(dst, ew):
    mesh = plsc.VectorSubcoreMesh(core_axis_name="c", subcore_axis_name="s")
    f = pl.kernel(
        _deg_body,
        out_type=jax.ShapeDtypeStruct((NTILES, N), jnp.float32),
        mesh=mesh,
        compiler_params=pltpu.CompilerParams(needs_layout_passes=False),
        scratch_types=[
            pltpu.VMEM((C_DEG,), jnp.int32),
            pltpu.VMEM((C_DEG,), jnp.float32),
            pltpu.VMEM((1, N), jnp.float32),
        ],
    )
    return f(dst, ew)


def _agg_body(hp_hbm, src_hbm, dst_hbm, ew_hbm, out_hbm,
              rows0, rows1, rows2, rows3, didx0, didx1, didx2, didx3,
              sidx_big, ew_big, shared,
              semi0, semi1, semi2, semi3,
              semg0, semg1, semg2, semg3,
              sems0, sems1, sems2, sems3):
    cid = lax.axis_index("c")
    sid = lax.axis_index("s")
    wid = sid * 2 + cid
    base = wid * TE

    rows = (rows0, rows1, rows2, rows3)
    didx = (didx0, didx1, didx2, didx3)
    semi = (semi0, semi1, semi2, semi3)
    semg = (semg0, semg1, semg2, semg3)
    sems = (sems0, sems1, sems2, sems3)

    zero = jnp.zeros((L,), jnp.float32)

    def zb(j, _):
        for k in range(D // L):
            rows0[j, pl.ds(k * L, L)] = zero
        return 0

    lax.fori_loop(0, CE, zb, 0)
    r0 = sid * RPT
    for k in range(RPT // CE):
        pltpu.sync_copy(rows0, shared.at[pl.ds(r0 + k * CE, CE)])
    rem = RPT - (RPT // CE) * CE
    pltpu.sync_copy(rows0.at[pl.ds(0, rem)],
                    shared.at[pl.ds(r0 + (RPT // CE) * CE, rem)])

    @pl.when(sid == 15)
    def _():
        pltpu.sync_copy(rows0.at[pl.ds(0, TAIL - RPT)],
                        shared.at[pl.ds(16 * RPT, TAIL - RPT)])

    plsc.subcore_barrier()

    pltpu.sync_copy(src_hbm.at[pl.ds(base, TE)], sidx_big)
    pltpu.sync_copy(ew_hbm.at[pl.ds(base, TE)], ew_big.at[pl.ds(0, TE)])

    last = (NCH - 1) * CE

    def issue_didx(b, c):
        off = base + jnp.minimum(c * CE, last)
        return pltpu.async_copy(dst_hbm.at[pl.ds(off, CE)], didx[b].at[0],
                                semi[b])

    def wait_didx(b):
        pltpu.make_async_copy(dst_hbm.at[pl.ds(0, CE)], didx[b].at[0],
                              semi[b]).wait()

    def issue_gather(b, c):
        off = jnp.minimum(c * CE, last)
        return pltpu.async_copy(hp_hbm.at[sidx_big.at[pl.ds(off, CE)]],
                                rows[b], semg[b])

    def wait_gather(b):
        pltpu.make_async_copy(hp_hbm.at[sidx_big.at[pl.ds(0, CE)]], rows[b],
                              semg[b]).wait()

    def scale(b, c):
        rb = rows[b]
        for g in range(CE // L):
            wv = ew_big[pl.ds(c * CE + g * L, L)]
            for u in range(L):
                wvec = jnp.full((L,), wv[u], jnp.float32)
                j = g * L + u
                for k in range(D // L):
                    rb[j, pl.ds(k * L, L)] = rb[j, pl.ds(k * L, L)] * wvec
        wv = ew_big[pl.ds(c * CE + (CE // L) * L, L)]
        for u in range(CE - (CE // L) * L):
            wvec = jnp.full((L,), wv[u], jnp.float32)
            j = (CE // L) * L + u
            for k in range(D // L):
                rb[j, pl.ds(k * L, L)] = rb[j, pl.ds(k * L, L)] * wvec

    def issue_scatter(b):
        return pltpu.async_copy(rows[b], shared.at[didx[b].at[0]], sems[b],
                                add=True)

    def wait_scatter(b):
        pltpu.make_async_copy(rows[b], shared.at[didx[b].at[0]],
                              sems[b]).wait()

    issue_didx(0, 0)
    issue_didx(1, 1)
    issue_gather(0, 0)
    issue_gather(1, 1)
    for b in range(2):
        wait_gather(b)
        scale(b, b)
        wait_didx(b)
        issue_scatter(b)
        issue_didx(b + 2, b + 2)
        issue_gather(b + 2, b + 2)

    def quad(p, _):
        c = p * 4 + 2
        for u in range(4):
            b = (2 + u) % 4
            bn = (b + 2) % 4
            wait_gather(b)
            scale(b, c + u)
            wait_didx(b)
            issue_scatter(b)
            wait_scatter(bn)
            issue_didx(bn, c + u + 2)
            issue_gather(bn, c + u + 2)
        return 0

    lax.fori_loop(0, (NCH - 2) // 4, quad, 0)
    wait_gather(NCH % 4)
    wait_gather((NCH + 1) % 4)
    wait_didx(NCH % 4)
    wait_didx((NCH + 1) % 4)
    wait_scatter((NCH - 2) % 4)
    wait_scatter((NCH - 1) % 4)

    plsc.subcore_barrier()

    pltpu.sync_copy(shared.at[pl.ds(r0, RPT)],
                    out_hbm.at[cid, pl.ds(r0, RPT)])

    @pl.when(sid == 15)
    def _():
        pltpu.sync_copy(shared.at[pl.ds(16 * RPT, TAIL - RPT)],
                        out_hbm.at[cid, pl.ds(16 * RPT, TAIL - RPT)])


def _sc_agg(hp, src, dst, ew):
    mesh = plsc.VectorSubcoreMesh(core_axis_name="c", subcore_axis_name="s")
    f = pl.kernel(
        _agg_body,
        out_type=jax.ShapeDtypeStruct((2, N, D), jnp.float32),
        mesh=mesh,
        compiler_params=pltpu.CompilerParams(needs_layout_passes=False),
        scratch_types=(
            [pltpu.VMEM((CE, D), jnp.float32)] * 4
            + [pltpu.VMEM((1, CE), jnp.int32)] * 4
            + [pltpu.VMEM((TE,), jnp.int32),
               pltpu.VMEM((TE + L,), jnp.float32),
               pltpu.VMEM_SHARED((N, D), jnp.float32)]
            + [pltpu.SemaphoreType.DMA] * 12
        ),
    )
    return f(hp, src, dst, ew)


def _tc_first_body(dp_ref, x_ref, W_ref, b_ref, hp_ref, dis_ref):
    ones = jnp.ones((NTILES, 1), jnp.float32)
    deg = lax.dot_general(dp_ref[...], ones, (((0,), (0,)), ((), ())),
                          preferred_element_type=jnp.float32) + 1.0
    dis = lax.rsqrt(deg)
    h = lax.dot_general(x_ref[...], W_ref[...], (((1,), (0,)), ((), ())),
                        preferred_element_type=jnp.float32)
    hp_ref[...] = (h + b_ref[...]) * dis
    dis_ref[...] = dis


def _tc_first(dp, x, W, brow):
    return pl.pallas_call(
        _tc_first_body,
        out_shape=(jax.ShapeDtypeStruct((N, D), jnp.float32),
                   jax.ShapeDtypeStruct((N, 1), jnp.float32)),
    )(dp, x, W, brow)


def _tc_mid_body(p2_ref, hp_ref, dis_ref, W_ref, b_ref, out_ref):
    dis = dis_ref[...]
    acc = p2_ref[0] + p2_ref[1]
    act = jnp.maximum(dis * (acc + hp_ref[...]), 0.0)
    h = lax.dot_general(act, W_ref[...], (((1,), (0,)), ((), ())),
                        preferred_element_type=jnp.float32)
    out_ref[...] = (h + b_ref[...]) * dis


def _tc_mid(p2, hp, dis, W, brow):
    return pl.pallas_call(
        _tc_mid_body,
        out_shape=jax.ShapeDtypeStruct((N, D), jnp.float32),
    )(p2, hp, dis, W, brow)


def _tc_last_body(p2_ref, hp_ref, dis_ref, out_ref):
    acc = p2_ref[0] + p2_ref[1]
    out_ref[...] = jnp.maximum(dis_ref[...] * (acc + hp_ref[...]), 0.0)


def _tc_last(p2, hp, dis):
    return pl.pallas_call(
        _tc_last_body,
        out_shape=jax.ShapeDtypeStruct((N, D), jnp.float32),
    )(p2, hp, dis)


def kernel(x, edge_index, edge_weight, W1, b1, W2, b2, W3, b3):
    src = edge_index[0]
    dst = edge_index[1]
    dp = _sc_deg(dst, edge_weight)
    h1p, dis = _tc_first(dp, x, W1, b1.reshape(1, D))
    p1 = _sc_agg(h1p, src, dst, edge_weight)
    h2p = _tc_mid(p1, h1p, dis, W2, b2.reshape(1, D))
    p2 = _sc_agg(h2p, src, dst, edge_weight)
    h3p = _tc_mid(p2, h2p, dis, W3, b3.reshape(1, D))
    p3 = _sc_agg(h3p, src, dst, edge_weight)
    return _tc_last(p3, h3p, dis)


# speedup vs baseline: 22.0794x; 1.0153x over previous
import jax
import jax.numpy as jnp
from jax import lax
from jax.experimental import pallas as pl
from jax.experimental.pallas import tpu as pltpu
from jax.experimental.pallas import tpu_sc as plsc

N = 10000
E = 320000
D = 128
NTILES = 32
L = 16
EC_DEG = E // NTILES
C_DEG = 2000
CE = 40
TE = E // NTILES
NCH = TE // CE
RPT = 624
TAIL = N - 15 * RPT
BN = 2048


def _deg_body(dst_hbm, ew_hbm, out_hbm, dst_v, ew_v, acc):
    wid = lax.axis_index("s") * 2 + lax.axis_index("c")
    zero = jnp.zeros((L,), jnp.float32)

    def zb(i, _):
        acc[0, pl.ds(i * L, L)] = zero
        return 0

    lax.fori_loop(0, N // L, zb, 0)

    base = wid * EC_DEG
    zrow = jnp.zeros((L,), jnp.int32)

    def chunk(c, _):
        off = base + c * C_DEG
        pltpu.sync_copy(dst_hbm.at[pl.ds(off, C_DEG)], dst_v)
        pltpu.sync_copy(ew_hbm.at[pl.ds(off, C_DEG)], ew_v)

        def ib(i, _):
            idx = dst_v[pl.ds(i * L, L)]
            w = ew_v[pl.ds(i * L, L)]
            plsc.addupdate_scatter(acc, [zrow, idx], w)
            return 0

        lax.fori_loop(0, C_DEG // L, ib, 0)
        return 0

    lax.fori_loop(0, EC_DEG // C_DEG, chunk, 0)
    pltpu.sync_copy(acc, out_hbm.at[pl.ds(wid, 1)])


def _sc_deg(dst, ew):
    mesh = plsc.VectorSubcoreMesh(core_axis_name="c", subcore_axis_name="s")
    f = pl.kernel(
        _deg_body,
        out_type=jax.ShapeDtypeStruct((NTILES, N), jnp.float32),
        mesh=mesh,
        compiler_params=pltpu.CompilerParams(needs_layout_passes=False),
        scratch_types=[
            pltpu.VMEM((C_DEG,), jnp.int32),
            pltpu.VMEM((C_DEG,), jnp.float32),
            pltpu.VMEM((1, N), jnp.float32),
        ],
    )
    return f(dst, ew)


def _agg_body(hp_hbm, src_hbm, dst_hbm, ew_hbm, out_hbm,
              rows0, rows1, rows2, rows3, didx0, didx1, didx2, didx3,
              sidx_big, ew_big, shared,
              semi0, semi1, semi2, semi3,
              semg0, semg1, semg2, semg3,
              sems0, sems1, sems2, sems3):
    cid = lax.axis_index("c")
    sid = lax.axis_index("s")
    wid = sid * 2 + cid
    base = wid * TE

    rows = (rows0, rows1, rows2, rows3)
    didx = (didx0, didx1, didx2, didx3)
    semi = (semi0, semi1, semi2, semi3)
    semg = (semg0, semg1, semg2, semg3)
    sems = (sems0, sems1, sems2, sems3)

    zero = jnp.zeros((L,), jnp.float32)

    def zb(j, _):
        for k in range(D // L):
            rows0[j, pl.ds(k * L, L)] = zero
        return 0

    lax.fori_loop(0, CE, zb, 0)

    hs = pltpu.async_copy(src_hbm.at[pl.ds(base, TE)], sidx_big, semg0)
    he = pltpu.async_copy(ew_hbm.at[pl.ds(base, TE)],
                          ew_big.at[pl.ds(0, TE)], semg1)

    r0 = sid * RPT
    zh = []
    for k in range(RPT // CE):
        zh.append(pltpu.async_copy(rows0, shared.at[pl.ds(r0 + k * CE, CE)],
                                   semi0))
    rem = RPT - (RPT // CE) * CE
    zh.append(pltpu.async_copy(rows0.at[pl.ds(0, rem)],
                               shared.at[pl.ds(r0 + (RPT // CE) * CE, rem)],
                               semi0))

    @pl.when(sid == 15)
    def _():
        pltpu.async_copy(rows0.at[pl.ds(0, TAIL - RPT)],
                         shared.at[pl.ds(16 * RPT, TAIL - RPT)],
                         semi0).wait()

    for h in zh:
        h.wait()
    hs.wait()
    he.wait()

    plsc.subcore_barrier()

    last = (NCH - 1) * CE

    def issue_didx(b, c):
        off = base + jnp.minimum(c * CE, last)
        return pltpu.async_copy(dst_hbm.at[pl.ds(off, CE)], didx[b].at[0],
                                semi[b])

    def wait_didx(b):
        pltpu.make_async_copy(dst_hbm.at[pl.ds(0, CE)], didx[b].at[0],
                              semi[b]).wait()

    def issue_gather(b, c):
        off = jnp.minimum(c * CE, last)
        return pltpu.async_copy(hp_hbm.at[sidx_big.at[pl.ds(off, CE)]],
                                rows[b], semg[b])

    def wait_gather(b):
        pltpu.make_async_copy(hp_hbm.at[sidx_big.at[pl.ds(0, CE)]], rows[b],
                              semg[b]).wait()

    def scale(b, c):
        rb = rows[b]
        for g in range(CE // L):
            wv = ew_big[pl.ds(c * CE + g * L, L)]
            for u in range(L):
                wvec = jnp.full((L,), wv[u], jnp.float32)
                j = g * L + u
                for k in range(D // L):
                    rb[j, pl.ds(k * L, L)] = rb[j, pl.ds(k * L, L)] * wvec
        wv = ew_big[pl.ds(c * CE + (CE // L) * L, L)]
        for u in range(CE - (CE // L) * L):
            wvec = jnp.full((L,), wv[u], jnp.float32)
            j = (CE // L) * L + u
            for k in range(D // L):
                rb[j, pl.ds(k * L, L)] = rb[j, pl.ds(k * L, L)] * wvec

    def issue_scatter(b):
        return pltpu.async_copy(rows[b], shared.at[didx[b].at[0]], sems[b],
                                add=True)

    def wait_scatter(b):
        pltpu.make_async_copy(rows[b], shared.at[didx[b].at[0]],
                              sems[b]).wait()

    issue_didx(0, 0)
    issue_didx(1, 1)
    issue_gather(0, 0)
    issue_gather(1, 1)
    for b in range(2):
        wait_gather(b)
        scale(b, b)
        wait_didx(b)
        issue_scatter(b)
        issue_didx(b + 2, b + 2)
        issue_gather(b + 2, b + 2)

    def quad(p, _):
        c = p * 4 + 2
        for u in range(4):
            b = (2 + u) % 4
            bn = (b + 2) % 4
            wait_gather(b)
            scale(b, c + u)
            wait_didx(b)
            issue_scatter(b)
            wait_scatter(bn)
            issue_didx(bn, c + u + 2)
            issue_gather(bn, c + u + 2)
        return 0

    lax.fori_loop(0, (NCH - 2) // 4, quad, 0)
    wait_gather(NCH % 4)
    wait_gather((NCH + 1) % 4)
    wait_didx(NCH % 4)
    wait_didx((NCH + 1) % 4)
    wait_scatter((NCH - 2) % 4)
    wait_scatter((NCH - 1) % 4)

    plsc.subcore_barrier()

    pltpu.sync_copy(shared.at[pl.ds(r0, RPT)],
                    out_hbm.at[cid, pl.ds(r0, RPT)])

    @pl.when(sid == 15)
    def _():
        pltpu.sync_copy(shared.at[pl.ds(16 * RPT, TAIL - RPT)],
                        out_hbm.at[cid, pl.ds(16 * RPT, TAIL - RPT)])


def _sc_agg(hp, src, dst, ew):
    mesh = plsc.VectorSubcoreMesh(core_axis_name="c", subcore_axis_name="s")
    f = pl.kernel(
        _agg_body,
        out_type=jax.ShapeDtypeStruct((2, N, D), jnp.float32),
        mesh=mesh,
        compiler_params=pltpu.CompilerParams(needs_layout_passes=False),
        scratch_types=(
            [pltpu.VMEM((CE, D), jnp.float32)] * 4
            + [pltpu.VMEM((1, CE), jnp.int32)] * 4
            + [pltpu.VMEM((TE,), jnp.int32),
               pltpu.VMEM((TE + L,), jnp.float32),
               pltpu.VMEM_SHARED((N, D), jnp.float32)]
            + [pltpu.SemaphoreType.DMA] * 12
        ),
    )
    return f(hp, src, dst, ew)


def _tc_first_body(dp_ref, x_ref, W_ref, b_ref, hp_ref, dis_ref):
    ones = jnp.ones((NTILES, 1), jnp.float32)
    deg = lax.dot_general(dp_ref[...], ones, (((0,), (0,)), ((), ())),
                          preferred_element_type=jnp.float32) + 1.0
    dis = lax.rsqrt(deg)
    h = lax.dot_general(x_ref[...], W_ref[...], (((1,), (0,)), ((), ())),
                        preferred_element_type=jnp.float32)
    hp_ref[...] = (h + b_ref[...]) * dis
    dis_ref[...] = dis


def _tc_first(dp, x, W, brow):
    return pl.pallas_call(
        _tc_first_body,
        grid=(pl.cdiv(N, BN),),
        in_specs=[
            pl.BlockSpec((NTILES, BN), lambda i: (0, i)),
            pl.BlockSpec((BN, D), lambda i: (i, 0)),
            pl.BlockSpec((D, D), lambda i: (0, 0)),
            pl.BlockSpec((1, D), lambda i: (0, 0)),
        ],
        out_specs=(pl.BlockSpec((BN, D), lambda i: (i, 0)),
                   pl.BlockSpec((BN, 1), lambda i: (i, 0))),
        out_shape=(jax.ShapeDtypeStruct((N, D), jnp.float32),
                   jax.ShapeDtypeStruct((N, 1), jnp.float32)),
    )(dp, x, W, brow)


def _tc_mid_body(p2_ref, hp_ref, dis_ref, W_ref, b_ref, out_ref):
    dis = dis_ref[...]
    acc = p2_ref[0] + p2_ref[1]
    act = jnp.maximum(dis * (acc + hp_ref[...]), 0.0)
    h = lax.dot_general(act, W_ref[...], (((1,), (0,)), ((), ())),
                        preferred_element_type=jnp.float32)
    out_ref[...] = (h + b_ref[...]) * dis


def _tc_mid(p2, hp, dis, W, brow):
    return pl.pallas_call(
        _tc_mid_body,
        grid=(pl.cdiv(N, BN),),
        in_specs=[
            pl.BlockSpec((2, BN, D), lambda i: (0, i, 0)),
            pl.BlockSpec((BN, D), lambda i: (i, 0)),
            pl.BlockSpec((BN, 1), lambda i: (i, 0)),
            pl.BlockSpec((D, D), lambda i: (0, 0)),
            pl.BlockSpec((1, D), lambda i: (0, 0)),
        ],
        out_specs=pl.BlockSpec((BN, D), lambda i: (i, 0)),
        out_shape=jax.ShapeDtypeStruct((N, D), jnp.float32),
    )(p2, hp, dis, W, brow)


def _tc_last_body(p2_ref, hp_ref, dis_ref, out_ref):
    acc = p2_ref[0] + p2_ref[1]
    out_ref[...] = jnp.maximum(dis_ref[...] * (acc + hp_ref[...]), 0.0)


def _tc_last(p2, hp, dis):
    return pl.pallas_call(
        _tc_last_body,
        grid=(pl.cdiv(N, BN),),
        in_specs=[
            pl.BlockSpec((2, BN, D), lambda i: (0, i, 0)),
            pl.BlockSpec((BN, D), lambda i: (i, 0)),
            pl.BlockSpec((BN, 1), lambda i: (i, 0)),
        ],
        out_specs=pl.BlockSpec((BN, D), lambda i: (i, 0)),
        out_shape=jax.ShapeDtypeStruct((N, D), jnp.float32),
    )(p2, hp, dis)


def kernel(x, edge_index, edge_weight, W1, b1, W2, b2, W3, b3):
    src = edge_index[0]
    dst = edge_index[1]
    dp = _sc_deg(dst, edge_weight)
    h1p, dis = _tc_first(dp, x, W1, b1.reshape(1, D))
    p1 = _sc_agg(h1p, src, dst, edge_weight)
    h2p = _tc_mid(p1, h1p, dis, W2, b2.reshape(1, D))
    p2 = _sc_agg(h2p, src, dst, edge_weight)
    h3p = _tc_mid(p2, h2p, dis, W3, b3.reshape(1, D))
    p3 = _sc_agg(h3p, src, dst, edge_weight)
    return _tc_last(p3, h3p, dis)
